# baseline (speedup 1.0000x reference)
import jax
import jax.numpy as jnp
from jax import lax
from jax.experimental import pallas as pl
from jax.experimental.pallas import tpu as pltpu
from jax.experimental.pallas import tpu_sc as plsc

_NUM_CORES = 2
_NUM_SUBCORES = 16
_NUM_WORKERS = _NUM_CORES * _NUM_SUBCORES

_NBUF = 2
_BLK = 3472
_NBLK = 9
_CHUNK = _BLK * _NBLK
_MAIN = _CHUNK * _NUM_WORKERS
_TAIL = 8


def _copy_body(table_hbm, out_hbm, bufs, rsems, wsems):
    wid = lax.axis_index("s") * _NUM_CORES + lax.axis_index("c")
    base = wid * _CHUNK

    def read(i):
        return pltpu.async_copy(
            table_hbm.at[pl.ds(base + i * _BLK, _BLK)],
            bufs[i % _NBUF],
            rsems[i % _NBUF],
        )

    def write(i):
        return pltpu.async_copy(
            bufs[i % _NBUF],
            out_hbm.at[pl.ds(base + i * _BLK, _BLK)],
            wsems[i % _NBUF],
        )

    rd = [None] * _NBLK
    wr = [None] * _NBLK
    for i in range(_NBUF):
        rd[i] = read(i)
    for i in range(_NBLK):
        rd[i].wait()
        wr[i] = write(i)
        if i + _NBUF < _NBLK:
            wr[i].wait()
            rd[i + _NBUF] = read(i + _NBUF)
    for i in range(max(_NBLK - _NBUF, 0), _NBLK):
        wr[i].wait()

    tail_workers = (table_hbm.shape[0] - _MAIN) // _TAIL

    @pl.when(wid < tail_workers)
    def _():
        tb = _MAIN + wid * _TAIL
        stage = bufs[0].at[pl.ds(0, _TAIL)]
        pltpu.sync_copy(table_hbm.at[pl.ds(tb, _TAIL)], stage)
        pltpu.sync_copy(stage, out_hbm.at[pl.ds(tb, _TAIL)])


def kernel(embed_node):
    mesh = plsc.VectorSubcoreMesh(
        core_axis_name="c", subcore_axis_name="s", num_cores=_NUM_CORES
    )
    fn = pl.kernel(
        _copy_body,
        out_type=jax.ShapeDtypeStruct(embed_node.shape, embed_node.dtype),
        mesh=mesh,
        scratch_types=[
            [pltpu.VMEM((_BLK, 16), jnp.float32) for _ in range(_NBUF)],
            [pltpu.SemaphoreType.DMA for _ in range(_NBUF)],
            [pltpu.SemaphoreType.DMA for _ in range(_NBUF)],
        ],
        compiler_params=pltpu.CompilerParams(use_tc_tiling_on_sc=False),
    )
    return fn(embed_node)


# speedup vs baseline: 14.8454x; 14.8454x over previous
import jax
import jax.numpy as jnp
from jax import lax
from jax.experimental import pallas as pl
from jax.experimental.pallas import tpu as pltpu
from jax.experimental.pallas import tpu_sc as plsc

_NUM_CORES = 2
_NUM_SUBCORES = 16
_NUM_WORKERS = _NUM_CORES * _NUM_SUBCORES

_ROWS = 16
_NBUF = 2
_BLKC = 3840
_NBLK = 8
_RESC = 512
_CHUNK = _BLKC * _NBLK + _RESC
_MAIN = _CHUNK * _NUM_WORKERS


def _copy_body(table_hbm, out_hbm, bufs, rsems, wsems):
    wid = lax.axis_index("s") * _NUM_CORES + lax.axis_index("c")
    base = wid * _CHUNK

    def read(i, off, width):
        return pltpu.async_copy(
            table_hbm.at[:, pl.ds(base + off, width)],
            bufs[i % _NBUF].at[:, pl.ds(0, width)],
            rsems[i % _NBUF],
        )

    def write(i, off, width):
        return pltpu.async_copy(
            bufs[i % _NBUF].at[:, pl.ds(0, width)],
            out_hbm.at[:, pl.ds(base + off, width)],
            wsems[i % _NBUF],
        )

    plan = [(j * _BLKC, _BLKC) for j in range(_NBLK)] + [(_NBLK * _BLKC, _RESC)]
    n = len(plan)
    rd = [None] * n
    wr = [None] * n
    for i in range(_NBUF):
        rd[i] = read(i, *plan[i])
    for i in range(n):
        rd[i].wait()
        wr[i] = write(i, *plan[i])
        if i + _NBUF < n:
            wr[i].wait()
            rd[i + _NBUF] = read(i + _NBUF, *plan[i + _NBUF])
    for i in range(max(n - _NBUF, 0), n):
        wr[i].wait()

    cols = table_hbm.shape[1]
    tail = cols - _MAIN
    full = tail // 128

    for w in range(full):
        off = _MAIN + w * 128

        @pl.when(wid == w)
        def _(off=off):
            stage = bufs[0].at[:, pl.ds(0, 128)]
            pltpu.sync_copy(table_hbm.at[:, pl.ds(off, 128)], stage)
            pltpu.sync_copy(stage, out_hbm.at[:, pl.ds(off, 128)])

    last = tail - full * 128
    if last:
        off = _MAIN + full * 128

        @pl.when(wid == full)
        def _():
            pltpu.sync_copy(table_hbm.at[:, pl.ds(off, last)], bufs[-1])
            pltpu.sync_copy(bufs[-1], out_hbm.at[:, pl.ds(off, last)])


def kernel(embed_node):
    xt = embed_node.T
    mesh = plsc.VectorSubcoreMesh(
        core_axis_name="c", subcore_axis_name="s", num_cores=_NUM_CORES
    )
    fn = pl.kernel(
        _copy_body,
        out_type=jax.ShapeDtypeStruct(xt.shape, xt.dtype),
        mesh=mesh,
        scratch_types=[
            [pltpu.VMEM((_ROWS, _BLKC), jnp.float32) for _ in range(_NBUF)]
            + [pltpu.VMEM((_ROWS, 64), jnp.float32)],
            [pltpu.SemaphoreType.DMA for _ in range(_NBUF)],
            [pltpu.SemaphoreType.DMA for _ in range(_NBUF)],
        ],
        compiler_params=pltpu.CompilerParams(use_tc_tiling_on_sc=True),
    )
    return fn(xt).T


# speedup vs baseline: 14.8590x; 1.0009x over previous
import jax
import jax.numpy as jnp
from jax import lax
from jax.experimental import pallas as pl
from jax.experimental.pallas import tpu as pltpu
from jax.experimental.pallas import tpu_sc as plsc

_NUM_CORES = 2
_NUM_SUBCORES = 16
_NUM_WORKERS = _NUM_CORES * _NUM_SUBCORES

_ROWS = 16
_NBUF = 2
_BLKC = 3840
_NBLK = 8
_RESC = 512
_CHUNK = _BLKC * _NBLK + _RESC
_MAIN = _CHUNK * _NUM_WORKERS


def _copy_body(table_hbm, out_hbm, bufs, rsems, wsems):
    wid = lax.axis_index("s") * _NUM_CORES + lax.axis_index("c")
    base = wid * _CHUNK

    def read(i, off, width):
        return pltpu.async_copy(
            table_hbm.at[:, pl.ds(base + off, width)],
            bufs[i % _NBUF].at[:, pl.ds(0, width)],
            rsems[i % _NBUF],
        )

    def write(i, off, width):
        return pltpu.async_copy(
            bufs[i % _NBUF].at[:, pl.ds(0, width)],
            out_hbm.at[:, pl.ds(base + off, width)],
            wsems[i % _NBUF],
        )

    plan = [(j * _BLKC, _BLKC) for j in range(_NBLK)] + [(_NBLK * _BLKC, _RESC)]
    n = len(plan)
    rd = [None] * n
    wr = [None] * n
    for i in range(_NBUF):
        rd[i] = read(i, *plan[i])
    for i in range(n):
        rd[i].wait()
        wr[i] = write(i, *plan[i])
        if i + _NBUF < n:
            wr[i].wait()
            rd[i + _NBUF] = read(i + _NBUF, *plan[i + _NBUF])
    for i in range(max(n - _NBUF, 0), n):
        wr[i].wait()

    cols = table_hbm.shape[1]
    tail = cols - _MAIN
    full = tail // 128

    for w in range(full):
        off = _MAIN + w * 128

        @pl.when(wid == w)
        def _(off=off):
            stage = bufs[0].at[:, pl.ds(0, 128)]
            pltpu.sync_copy(table_hbm.at[:, pl.ds(off, 128)], stage)
            pltpu.sync_copy(stage, out_hbm.at[:, pl.ds(off, 128)])

    last = tail - full * 128
    if last:
        off = _MAIN + full * 128

        @pl.when(wid == full)
        def _():
            pltpu.sync_copy(table_hbm.at[:, pl.ds(off, last)], bufs[-1])
            pltpu.sync_copy(bufs[-1], out_hbm.at[:, pl.ds(off, last)])


def kernel(embed_node):
    xt = embed_node.T
    mesh = plsc.VectorSubcoreMesh(
        core_axis_name="c", subcore_axis_name="s", num_cores=_NUM_CORES
    )
    fn = pl.kernel(
        _copy_body,
        out_type=jax.ShapeDtypeStruct(xt.shape, xt.dtype),
        mesh=mesh,
        scratch_types=[
            [pltpu.VMEM((_ROWS, _BLKC), jnp.float32) for _ in range(_NBUF)]
            + [pltpu.VMEM((_ROWS, 64), jnp.float32)],
            [pltpu.SemaphoreType.DMA for _ in range(_NBUF)],
            [pltpu.SemaphoreType.DMA for _ in range(_NBUF)],
        ],
        compiler_params=pltpu.CompilerParams(
            use_tc_tiling_on_sc=True, skip_device_barrier=True
        ),
    )
    return fn(xt).T


# speedup vs baseline: 14.8669x; 1.0005x over previous
import jax
import jax.numpy as jnp
from jax import lax
from jax.experimental import pallas as pl
from jax.experimental.pallas import tpu as pltpu
from jax.experimental.pallas import tpu_sc as plsc

_NUM_CORES = 2
_NUM_SUBCORES = 16
_NUM_WORKERS = _NUM_CORES * _NUM_SUBCORES

_ROWS = 16
_NBUF = 3
_BLKC = 2560
_NBLK = 12
_RESC = 512
_CHUNK = _BLKC * _NBLK + _RESC
_MAIN = _CHUNK * _NUM_WORKERS


def _copy_body(table_hbm, out_hbm, bufs, rsems, wsems):
    wid = lax.axis_index("s") * _NUM_CORES + lax.axis_index("c")
    base = wid * _CHUNK

    def read(i, off, width):
        return pltpu.async_copy(
            table_hbm.at[:, pl.ds(base + off, width)],
            bufs[i % _NBUF].at[:, pl.ds(0, width)],
            rsems[i % _NBUF],
        )

    def write(i, off, width):
        return pltpu.async_copy(
            bufs[i % _NBUF].at[:, pl.ds(0, width)],
            out_hbm.at[:, pl.ds(base + off, width)],
            wsems[i % _NBUF],
        )

    plan = [(j * _BLKC, _BLKC) for j in range(_NBLK)] + [(_NBLK * _BLKC, _RESC)]
    n = len(plan)
    rd = [None] * n
    wr = [None] * n
    for i in range(_NBUF):
        rd[i] = read(i, *plan[i])
    for i in range(n):
        rd[i].wait()
        wr[i] = write(i, *plan[i])
        if i + _NBUF < n:
            wr[i].wait()
            rd[i + _NBUF] = read(i + _NBUF, *plan[i + _NBUF])
    for i in range(max(n - _NBUF, 0), n):
        wr[i].wait()

    cols = table_hbm.shape[1]
    tail = cols - _MAIN
    full = tail // 128

    for w in range(full):
        off = _MAIN + w * 128

        @pl.when(wid == w)
        def _(off=off):
            stage = bufs[0].at[:, pl.ds(0, 128)]
            pltpu.sync_copy(table_hbm.at[:, pl.ds(off, 128)], stage)
            pltpu.sync_copy(stage, out_hbm.at[:, pl.ds(off, 128)])

    last = tail - full * 128
    if last:
        off = _MAIN + full * 128

        @pl.when(wid == full)
        def _():
            pltpu.sync_copy(table_hbm.at[:, pl.ds(off, last)], bufs[-1])
            pltpu.sync_copy(bufs[-1], out_hbm.at[:, pl.ds(off, last)])


def kernel(embed_node):
    xt = embed_node.T
    mesh = plsc.VectorSubcoreMesh(
        core_axis_name="c", subcore_axis_name="s", num_cores=_NUM_CORES
    )
    fn = pl.kernel(
        _copy_body,
        out_type=jax.ShapeDtypeStruct(xt.shape, xt.dtype),
        mesh=mesh,
        scratch_types=[
            [pltpu.VMEM((_ROWS, _BLKC), jnp.float32) for _ in range(_NBUF)]
            + [pltpu.VMEM((_ROWS, 64), jnp.float32)],
            [pltpu.SemaphoreType.DMA for _ in range(_NBUF)],
            [pltpu.SemaphoreType.DMA for _ in range(_NBUF)],
        ],
        compiler_params=pltpu.CompilerParams(
            use_tc_tiling_on_sc=True
        ),
    )
    return fn(xt).T
